# baseline (speedup 1.0000x reference)
import functools

import jax
import jax.numpy as jnp
from jax import lax
from jax.experimental import pallas as pl
from jax.experimental.pallas import tpu as pltpu
from jax.experimental.pallas import tpu_sc as plsc


E, I, J = 64, 1024, 1024
B, T, K = 1, 8192, 32
N = E * K

_C = 2
_EC = E // _C
_NC = _EC * K


def _sc_gather_chunk(table, idx):
  info = plsc.get_sparse_core_info()
  nw = info.num_cores * info.num_subcores
  b_per_w = _NC // nw
  mesh = plsc.VectorSubcoreMesh(core_axis_name="c", subcore_axis_name="s")

  @functools.partial(
      pl.kernel,
      mesh=mesh,
      out_type=jax.ShapeDtypeStruct((_NC, I), jnp.float32),
      scratch_types=[
          pltpu.VMEM((b_per_w,), jnp.int32),
          pltpu.VMEM((b_per_w, I), jnp.float32),
          pltpu.SemaphoreType.DMA,
      ],
  )
  def k(table_hbm, idx_hbm, out_hbm, idx_v, rows_v, sem):
    wid = lax.axis_index("s") * info.num_cores + lax.axis_index("c")
    base = wid * b_per_w
    pltpu.sync_copy(idx_hbm.at[pl.ds(base, b_per_w)], idx_v)
    pltpu.async_copy(table_hbm.at[idx_v], rows_v, sem).wait()
    pltpu.sync_copy(rows_v, out_hbm.at[pl.ds(base, b_per_w)])

  return k(table, idx)


_NCH = 4
_IC = I // _NCH
_NSLOT = 8
_GC = _EC * _NCH


def _chunk_copy(w_hbm, w_bufs, sems, ebase, g, slot):
  e = g // _NCH
  c = lax.rem(g, _NCH)
  return pltpu.make_async_copy(
      w_hbm.at[ebase + e, pl.ds(c * _IC, _IC)], w_bufs.at[slot], sems.at[slot]
  )


def _mm_body(ebase, *refs):
  if len(refs) == 6:
    _, xg_ref, w_hbm, out_ref, w_bufs, sems = refs
  else:
    xg_ref, w_hbm, out_ref, w_bufs, sems = refs
  e = pl.program_id(0)

  @pl.when(e == 0)
  def _prime():
    for b in range(_NSLOT):
      _chunk_copy(w_hbm, w_bufs, sems, ebase, b, b).start()

  for c in range(_NCH):
    g = e * _NCH + c
    slot = lax.rem(g, _NSLOT)
    _chunk_copy(w_hbm, w_bufs, sems, ebase, g, slot).wait()
    part = jnp.dot(
        xg_ref[e, :, c * _IC:(c + 1) * _IC],
        w_bufs[slot],
        preferred_element_type=jnp.float32,
    )
    if c == 0:
      out_ref[0] = part
    else:
      out_ref[0] += part
    ng = g + _NSLOT

    @pl.when(ng < _GC)
    def _refill():
      _chunk_copy(w_hbm, w_bufs, sems, ebase, ng, slot).start()


def _tc_matmul_chunk(y_acc, xg, w, ebase):
  xg_spec = pl.BlockSpec((_EC, K, I), lambda e: (0, 0, 0))
  any_spec = pl.BlockSpec(memory_space=pl.ANY)
  if y_acc is None:
    in_specs, args, aliases = [xg_spec, any_spec], (xg, w), {}
  else:
    in_specs = [any_spec, xg_spec, any_spec]
    args, aliases = (y_acc, xg, w), {0: 0}
  return pl.pallas_call(
      functools.partial(_mm_body, ebase),
      grid=(_EC,),
      in_specs=in_specs,
      out_specs=pl.BlockSpec((1, K, J), lambda e: (ebase + e, 0, 0)),
      out_shape=jax.ShapeDtypeStruct((E, K, J), jnp.float32),
      scratch_shapes=[
          pltpu.VMEM((_NSLOT, _IC, J), jnp.float32),
          pltpu.SemaphoreType.DMA((_NSLOT,)),
      ],
      input_output_aliases=aliases,
  )(*args)


@jax.jit
def kernel(X, ind, W):
  table = X.reshape(T, I)
  idx = ind.reshape(N).astype(jnp.int32)
  xgs = [
      _sc_gather_chunk(table, idx[c * _NC:(c + 1) * _NC]) for c in range(_C)
  ]
  y = None
  for c in range(_C):
    y = _tc_matmul_chunk(y, xgs[c].reshape(_EC, K, I), W, c * _EC)
  return y.reshape(B, E, K, J)


# speedup vs baseline: 1.0191x; 1.0191x over previous
import functools

import jax
import jax.numpy as jnp
from jax import lax
from jax.experimental import pallas as pl
from jax.experimental.pallas import tpu as pltpu
from jax.experimental.pallas import tpu_sc as plsc


E, I, J = 64, 1024, 1024
B, T, K = 1, 8192, 32
N = E * K


def _sc_gather(table, idx):
  info = plsc.get_sparse_core_info()
  nw = info.num_cores * info.num_subcores
  b_per_w = N // nw
  mesh = plsc.VectorSubcoreMesh(core_axis_name="c", subcore_axis_name="s")

  @functools.partial(
      pl.kernel,
      mesh=mesh,
      out_type=jax.ShapeDtypeStruct((N, I), jnp.float32),
      scratch_types=[
          pltpu.VMEM((b_per_w,), jnp.int32),
          pltpu.VMEM((b_per_w, I), jnp.float32),
          pltpu.SemaphoreType.DMA,
          pltpu.SemaphoreType.DMA,
          pltpu.SemaphoreType.DMA,
          pltpu.SemaphoreType.DMA,
      ],
  )
  def k(table_hbm, idx_hbm, out_hbm, idx_v, rows_v, sg0, sg1, sw0, sw1):
    wid = lax.axis_index("s") * info.num_cores + lax.axis_index("c")
    base = wid * b_per_w
    h = b_per_w // 2
    pltpu.sync_copy(idx_hbm.at[pl.ds(base, b_per_w)], idx_v)
    g0 = pltpu.async_copy(
        table_hbm.at[idx_v.at[pl.ds(0, h)]], rows_v.at[pl.ds(0, h)], sg0
    )
    g1 = pltpu.async_copy(
        table_hbm.at[idx_v.at[pl.ds(h, h)]], rows_v.at[pl.ds(h, h)], sg1
    )
    g0.wait()
    w0 = pltpu.async_copy(
        rows_v.at[pl.ds(0, h)], out_hbm.at[pl.ds(base, h)], sw0
    )
    g1.wait()
    w1 = pltpu.async_copy(
        rows_v.at[pl.ds(h, h)], out_hbm.at[pl.ds(base + h, h)], sw1
    )
    w0.wait()
    w1.wait()

  return k(table, idx)


_NBUF = 4


def _mm_body(xg_ref, w_hbm, out_ref, w_bufs, sems):
  t = pl.program_id(0)

  @pl.when(t == 0)
  def _prime():
    for b in range(_NBUF):
      pltpu.make_async_copy(w_hbm.at[b], w_bufs.at[b], sems.at[b]).start()

  @pl.when(t > 0)
  def _compute():
    ep = t - 1
    slotp = lax.rem(ep, _NBUF)
    out_ref[0] = jnp.dot(
        xg_ref[ep], w_bufs[slotp], preferred_element_type=jnp.float32
    )
    nxt = ep + _NBUF

    @pl.when(nxt < E)
    def _refill():
      pltpu.make_async_copy(
          w_hbm.at[nxt], w_bufs.at[slotp], sems.at[slotp]
      ).start()

  @pl.when(t < E)
  def _await():
    slot = lax.rem(t, _NBUF)
    pltpu.make_async_copy(w_hbm.at[t], w_bufs.at[slot], sems.at[slot]).wait()


def _tc_matmul(xg, w):
  return pl.pallas_call(
      _mm_body,
      grid=(E + 1,),
      in_specs=[
          pl.BlockSpec((E, K, I), lambda t: (0, 0, 0)),
          pl.BlockSpec(memory_space=pl.ANY),
      ],
      out_specs=pl.BlockSpec(
          (1, K, J), lambda t: (jnp.maximum(t - 1, 0), 0, 0)
      ),
      out_shape=jax.ShapeDtypeStruct((E, K, J), jnp.float32),
      scratch_shapes=[
          pltpu.VMEM((_NBUF, I, J), jnp.float32),
          pltpu.SemaphoreType.DMA((_NBUF,)),
      ],
  )(xg, w)


@jax.jit
def kernel(X, ind, W):
  table = X.reshape(T, I)
  idx = ind.reshape(N).astype(jnp.int32)
  xg = _sc_gather(table, idx)
  y = _tc_matmul(xg.reshape(E, K, I), W)
  return y.reshape(B, E, K, J)
